# baseline (speedup 1.0000x reference)
import numpy as np
import jax
import jax.numpy as jnp
from jax.experimental import pallas as pl
from jax.experimental.pallas import tpu as pltpu

_N = 1024
_B = 8
_SLAB = 128
_Q = 256

def _build_h(n: int) -> np.ndarray:
    i = np.arange(n)
    parity = np.array([bin(v).count("1") & 1 for v in range(n)], dtype=np.int8)
    return (1.0 - 2.0 * parity[i[:, None] & i[None, :]]).astype(np.float32)

_H256 = _build_h(_Q)


def _fused_body(x_ref, h_ref, o_ref, m_ref):
    i = pl.program_id(0)
    h256 = h_ref[...]

    @pl.when(i < _B)
    def _right():
        a = x_ref[...].astype(jnp.bfloat16)
        a2 = a.reshape(_B, _SLAB, _N).reshape(_B * _SLAB, _N)
        lo, hi = a2[:, :512], a2[:, 512:]
        v0, v1 = lo + hi, lo - hi
        w = [v0[:, :_Q] + v0[:, _Q:], v0[:, :_Q] - v0[:, _Q:],
             v1[:, :_Q] + v1[:, _Q:], v1[:, :_Q] - v1[:, _Q:]]
        c = jnp.concatenate(
            [jnp.dot(wb, h256, preferred_element_type=jnp.float32)
             for wb in w], axis=1)
        m_ref[pl.ds(i * _N, _N), :] = c.astype(jnp.bfloat16)

    @pl.when(jnp.logical_and(i >= _B, i < _B + 2))
    def _left_h4():
        p = i - _B
        rows = [m_ref[pl.ds((2 * rh + p) * _N, _N), :] for rh in range(4)]
        t0, t1 = rows[0] + rows[2], rows[1] + rows[3]
        t2, t3 = rows[0] - rows[2], rows[1] - rows[3]
        u = [t0 + t1, t0 - t1, t2 + t3, t2 - t3]
        for rh in range(4):
            m_ref[pl.ds((2 * rh + p) * _N, _N), :] = u[rh]

    @pl.when(i >= _B + 2)
    def _left():
        k = i - (_B + 2)
        rh = k // 2
        hs = jnp.where((k % 2) == 0, h256[:_SLAB, :], h256[_SLAB:, :])
        outs = []
        for b in range(_B):
            cb = jnp.concatenate(
                [m_ref[pl.ds((2 * rh + j) * _N + b * _SLAB, _SLAB), :]
                 for j in range(2)], axis=0)
            outs.append(jnp.dot(hs, cb, preferred_element_type=jnp.float32))
        st = jnp.stack(outs, axis=0)
        o_ref[...] = st.reshape(_B, _SLAB * _N)


def kernel(x, signs, indxs):
    b, dim = x.shape
    h16 = jnp.asarray(_H256 * (1.0 / 32.0), dtype=jnp.bfloat16)
    out = pl.pallas_call(
        _fused_body,
        grid=(2 * _B + 2,),
        in_specs=[
            pl.BlockSpec((_B, _SLAB * _N), lambda i: (0, jnp.minimum(i, _B - 1))),
            pl.BlockSpec((_Q, _Q), lambda i: (0, 0)),
        ],
        out_specs=pl.BlockSpec((_B, _SLAB * _N),
                               lambda i: (0, jnp.maximum(i - (_B + 2), 0))),
        out_shape=jax.ShapeDtypeStruct((b, dim), jnp.float32),
        scratch_shapes=[pltpu.VMEM((_B * _N, _N), jnp.bfloat16)],
    )(x, h16)
    return out


# speedup vs baseline: 1.0200x; 1.0200x over previous
import numpy as np
import jax
import jax.numpy as jnp
from jax.experimental import pallas as pl
from jax.experimental.pallas import tpu as pltpu

_N = 1024
_B = 8
_SLAB = 128
_Q = 256

def _build_h(n: int) -> np.ndarray:
    i = np.arange(n)
    parity = np.array([bin(v).count("1") & 1 for v in range(n)], dtype=np.int8)
    return (1.0 - 2.0 * parity[i[:, None] & i[None, :]]).astype(np.float32)

_H256 = _build_h(_Q)


def _fused_body(x_ref, h_ref, o_ref, m_ref):
    i = pl.program_id(0)
    h256 = h_ref[...]

    @pl.when(i < _B)
    def _right():
        a = x_ref[...]
        a2 = (a.reshape(_B, _SLAB, _N)
               .reshape(_B * _SLAB, _N).astype(jnp.bfloat16))
        lo, hi = a2[:, :512], a2[:, 512:]
        v0, v1 = lo + hi, lo - hi
        w = [v0[:, :_Q] + v0[:, _Q:], v0[:, :_Q] - v0[:, _Q:],
             v1[:, :_Q] + v1[:, _Q:], v1[:, :_Q] - v1[:, _Q:]]
        c = jnp.concatenate(
            [jnp.dot(wb, h256, preferred_element_type=jnp.float32)
             for wb in w], axis=1)
        m_ref[pl.ds(i * _N, _N), :] = c.astype(jnp.bfloat16)

    @pl.when(jnp.logical_and(i >= _B - 1, i < _B + 1))
    def _left_h4():
        p = i - (_B - 1)
        rows = [m_ref[pl.ds((2 * rh + p) * _N, _N), :] for rh in range(4)]
        t0, t1 = rows[0] + rows[2], rows[1] + rows[3]
        t2, t3 = rows[0] - rows[2], rows[1] - rows[3]
        u = [t0 + t1, t0 - t1, t2 + t3, t2 - t3]
        for rh in range(4):
            m_ref[pl.ds((2 * rh + p) * _N, _N), :] = u[rh]

    @pl.when(i >= _B + 1)
    def _left():
        k = i - (_B + 1)
        rh = k // 2
        hs = jnp.where((k % 2) == 0, h256[:_SLAB, :], h256[_SLAB:, :])
        outs = []
        for b in range(_B):
            cb = jnp.concatenate(
                [m_ref[pl.ds((2 * rh + j) * _N + b * _SLAB, _SLAB), :]
                 for j in range(2)], axis=0)
            outs.append(jnp.dot(hs, cb, preferred_element_type=jnp.float32))
        st = jnp.stack(outs, axis=0)
        o_ref[...] = st.reshape(_B, _SLAB * _N)


def kernel(x, signs, indxs):
    b, dim = x.shape
    h16 = jnp.asarray(_H256 * (1.0 / 32.0), dtype=jnp.bfloat16)
    out = pl.pallas_call(
        _fused_body,
        grid=(2 * _B + 1,),
        in_specs=[
            pl.BlockSpec((_B, _SLAB * _N), lambda i: (0, jnp.minimum(i, _B - 1))),
            pl.BlockSpec((_Q, _Q), lambda i: (0, 0)),
        ],
        out_specs=pl.BlockSpec((_B, _SLAB * _N),
                               lambda i: (0, jnp.maximum(i - (_B + 1), 0))),
        out_shape=jax.ShapeDtypeStruct((b, dim), jnp.float32),
        scratch_shapes=[pltpu.VMEM((_B * _N, _N), jnp.bfloat16)],
    )(x, h16)
    return out


# speedup vs baseline: 1.0239x; 1.0038x over previous
import numpy as np
import jax
import jax.numpy as jnp
from jax.experimental import pallas as pl
from jax.experimental.pallas import tpu as pltpu

_N = 1024
_B = 8
_SLAB = 128
_Q = 256

def _build_h(n: int) -> np.ndarray:
    i = np.arange(n)
    parity = np.array([bin(v).count("1") & 1 for v in range(n)], dtype=np.int8)
    return (1.0 - 2.0 * parity[i[:, None] & i[None, :]]).astype(np.float32)

_H256 = _build_h(_Q)


def _fused_body(x_ref, h_ref, o_ref, m_ref):
    i = pl.program_id(0)
    h256 = h_ref[...]

    @pl.when(i < _B)
    def _right():
        a = x_ref[...]
        a2 = (a.reshape(_B, _SLAB, _N)
               .reshape(_B * _SLAB, _N).astype(jnp.bfloat16))
        lo, hi = a2[:, :512], a2[:, 512:]
        v0, v1 = lo + hi, lo - hi
        w = [v0[:, :_Q] + v0[:, _Q:], v0[:, :_Q] - v0[:, _Q:],
             v1[:, :_Q] + v1[:, _Q:], v1[:, :_Q] - v1[:, _Q:]]
        for ch in range(4):
            cb = jnp.dot(w[ch], h256, preferred_element_type=jnp.float32)
            m_ref[pl.ds(i * _N, _N), ch * _Q:(ch + 1) * _Q] = (
                cb.astype(jnp.bfloat16))

    @pl.when(jnp.logical_and(i >= _B - 1, i < _B + 1))
    def _left_h4():
        p = i - (_B - 1)
        rows = [m_ref[pl.ds((2 * rh + p) * _N, _N), :] for rh in range(4)]
        t0, t1 = rows[0] + rows[2], rows[1] + rows[3]
        t2, t3 = rows[0] - rows[2], rows[1] - rows[3]
        u = [t0 + t1, t0 - t1, t2 + t3, t2 - t3]
        for rh in range(4):
            m_ref[pl.ds((2 * rh + p) * _N, _N), :] = u[rh]

    @pl.when(i >= _B + 1)
    def _left():
        k = i - (_B + 1)
        rh = k // 2
        hs = jnp.where((k % 2) == 0, h256[:_SLAB, :], h256[_SLAB:, :])
        outs = []
        for b in range(_B):
            cb = jnp.concatenate(
                [m_ref[pl.ds((2 * rh + j) * _N + b * _SLAB, _SLAB), :]
                 for j in range(2)], axis=0)
            outs.append(jnp.dot(hs, cb, preferred_element_type=jnp.float32))
        st = jnp.stack(outs, axis=0)
        o_ref[...] = st.reshape(_B, _SLAB * _N)


def kernel(x, signs, indxs):
    b, dim = x.shape
    h16 = jnp.asarray(_H256 * (1.0 / 32.0), dtype=jnp.bfloat16)
    out = pl.pallas_call(
        _fused_body,
        grid=(2 * _B + 1,),
        in_specs=[
            pl.BlockSpec((_B, _SLAB * _N), lambda i: (0, jnp.minimum(i, _B - 1))),
            pl.BlockSpec((_Q, _Q), lambda i: (0, 0)),
        ],
        out_specs=pl.BlockSpec((_B, _SLAB * _N),
                               lambda i: (0, jnp.maximum(i - (_B + 1), 0))),
        out_shape=jax.ShapeDtypeStruct((b, dim), jnp.float32),
        scratch_shapes=[pltpu.VMEM((_B * _N, _N), jnp.bfloat16)],
    )(x, h16)
    return out
